# baseline (speedup 1.0000x reference)
import functools

import jax
import jax.numpy as jnp
from jax import lax
from jax.experimental import pallas as pl
from jax.experimental.pallas import tpu as pltpu
from jax.experimental.pallas import tpu_sc as plsc

_N, _D = 8192, 2048
_NC, _NS, _L = 2, 16, 16
_NW = _NC * _NS
_K = 8
_NB = 3
_UNROLL = 8
_VPR = _D // _L

_S = 3584
_TCBLK = 512


def _make_sc_body(split):
    rpw = (_N - split) // _NW
    nvr = rpw // _L

    def _sc_body(x_hbm, t_hbm, q_hbm, out_hbm,
                 q_v, idx_v, qs_v, xb0, tb0, xb1, tb1, xb2, tb2, accb,
                 sx0, st0, sx1, st1, sx2, st2):
        wid = lax.axis_index("s") * _NC + lax.axis_index("c")
        base = split + wid * rpw
        pltpu.sync_copy(q_hbm.at[pl.ds(base, rpw)], q_v)

        def _pad(j, _):
            sl = pl.ds(j * _L, _L)
            idx_v[sl] = jnp.full((_L,), base, jnp.int32)
            qs_v[sl] = jnp.zeros((_L,), jnp.int32)
            return 0

        lax.fori_loop(0, nvr + 1, _pad, 0)

        def _compact(j, cnt):
            qv = q_v[pl.ds(j * _L, _L)]
            m = qv != 0
            mi = jnp.where(m, 1, 0)
            pos = jnp.maximum(cnt + plsc.cumsum(mi) - 1, 0)
            rows = lax.iota(jnp.int32, _L) + (base + j * _L)
            plsc.store_scatter(idx_v, [pos], rows, mask=m)
            plsc.store_scatter(qs_v, [pos], qv, mask=m)
            return cnt + jnp.sum(mi)

        cnt = lax.fori_loop(0, nvr, _compact, jnp.int32(0))

        nch = (cnt + _K - 1) // _K
        ngrp = (nch + _NB - 1) // _NB

        bufs = ((xb0, tb0, sx0, st0), (xb1, tb1, sx1, st1), (xb2, tb2, sx2, st2))

        def issue(c, b):
            off = pl.multiple_of(c * _K, _K)
            idxs = idx_v.at[pl.ds(off, _K)]
            xb, tb, sx, st = bufs[b]
            pltpu.async_copy(x_hbm.at[idxs], xb, sx)
            pltpu.async_copy(t_hbm.at[idxs], tb, st)

        def wait(b):
            xb, tb, sx, st = bufs[b]
            dummy = idx_v.at[pl.ds(0, _K)]
            pltpu.make_async_copy(x_hbm.at[dummy], xb, sx).wait()
            pltpu.make_async_copy(t_hbm.at[dummy], tb, st).wait()

        def compute(c, b, acc):
            off = pl.multiple_of(c * _K, _K)
            xb, tb, _, _ = bufs[b]

            def row(k, a0):
                qb = plsc.load_gather(qs_v, [jnp.full((_L,), off + k, jnp.int32)])
                m1 = qb == 1
                g2 = jnp.where(qb == 2, 1.0, 0.0)

                def inner(v, a):
                    for u in range(_UNROLL):
                        sl = pl.ds((v * _UNROLL + u) * _L, _L)
                        x = xb[k, sl]
                        t = tb[k, sl]
                        d = x - t
                        w = jnp.where(m1, t, g2)
                        g = w * d
                        a = a + g * g
                    return a

                return lax.fori_loop(0, _VPR // _UNROLL, inner, a0)

            return lax.fori_loop(0, _K, row, acc)

        @pl.when(nch < _NB)
        def _zero_bufs():
            zk = jnp.zeros((_L,), jnp.float32)
            for xb, tb, _, _ in bufs:
                def _z(w, _, _xb=xb, _tb=tb):
                    k = w // _VPR
                    sl = pl.ds((w % _VPR) * _L, _L)
                    _xb[k, sl] = zk
                    _tb[k, sl] = zk
                    return 0

                lax.fori_loop(0, _K * _VPR, _z, 0)

        for b in range(_NB):
            @pl.when(b < nch)
            def _prime(b=b):
                issue(b, b)

        def group(i, acc):
            c0 = i * _NB
            for b in range(_NB):
                c = c0 + b

                @pl.when(c < nch)
                def _w(b=b):
                    wait(b)

                acc = compute(c, b, acc)

                @pl.when(c + _NB < nch)
                def _i(c=c, b=b):
                    issue(c + _NB, b)
            return acc

        acc = lax.fori_loop(0, ngrp, group, jnp.zeros((_L,), jnp.float32))

        accb[...] = acc
        pltpu.sync_copy(accb, out_hbm.at[wid])

    return _sc_body, rpw


def _sc_partials(input, target, q, split):
    body, rpw = _make_sc_body(split)
    ipad = rpw + _L
    mesh = plsc.VectorSubcoreMesh(core_axis_name="c", subcore_axis_name="s")
    f = functools.partial(
        pl.kernel,
        mesh=mesh,
        compiler_params=pltpu.CompilerParams(needs_layout_passes=False),
        out_type=jax.ShapeDtypeStruct((_NW, _L), jnp.float32),
        scratch_types=[
            pltpu.VMEM((rpw,), jnp.int32),
            pltpu.VMEM((ipad,), jnp.int32),
            pltpu.VMEM((ipad,), jnp.int32),
            pltpu.VMEM((_K, _D), jnp.float32),
            pltpu.VMEM((_K, _D), jnp.float32),
            pltpu.VMEM((_K, _D), jnp.float32),
            pltpu.VMEM((_K, _D), jnp.float32),
            pltpu.VMEM((_K, _D), jnp.float32),
            pltpu.VMEM((_K, _D), jnp.float32),
            pltpu.VMEM((_L,), jnp.float32),
            pltpu.SemaphoreType.DMA,
            pltpu.SemaphoreType.DMA,
            pltpu.SemaphoreType.DMA,
            pltpu.SemaphoreType.DMA,
            pltpu.SemaphoreType.DMA,
            pltpu.SemaphoreType.DMA,
        ],
    )(body)
    return f(input, target, q)


def _tc_body(q_ref, x_ref, t_ref, out_ref):
    i = pl.program_id(0)

    @pl.when(i == 0)
    def _init():
        out_ref[...] = jnp.zeros_like(out_ref)

    x = x_ref[...]
    t = t_ref[...]
    qb = q_ref[...]
    d = x - t
    w = jnp.where(qb == 1, t, jnp.where(qb == 2, 1.0, 0.0).astype(jnp.float32))
    g = w * d
    out_ref[...] += jnp.sum(g * g).reshape(1, 1)


def _tc_sum(input, target, q, split):
    q2 = q[:, None]
    return pl.pallas_call(
        _tc_body,
        grid=(split // _TCBLK,),
        in_specs=[
            pl.BlockSpec((_TCBLK, 1), lambda i: (i, 0)),
            pl.BlockSpec((_TCBLK, _D), lambda i: (i, 0)),
            pl.BlockSpec((_TCBLK, _D), lambda i: (i, 0)),
        ],
        out_specs=pl.BlockSpec((1, 1), lambda i: (0, 0)),
        out_shape=jax.ShapeDtypeStruct((1, 1), jnp.float32),
    )(q2, input, target)


def kernel(input, target, q):
    sc = _sc_partials(input, target, q, _S)
    tc = _tc_sum(input, target, q, _S)
    return (tc[0, 0] + jnp.sum(sc)) / (_N * _D)


# speedup vs baseline: 1.0053x; 1.0053x over previous
import functools

import jax
import jax.numpy as jnp
from jax import lax
from jax.experimental import pallas as pl
from jax.experimental.pallas import tpu as pltpu
from jax.experimental.pallas import tpu_sc as plsc

_N, _D = 8192, 2048
_NC, _NS, _L = 2, 16, 16
_NW = _NC * _NS
_K = 8
_NB = 3
_UNROLL = 8
_VPR = _D // _L

_S = 3072
_TCBLK = 512


def _make_sc_body(split):
    rpw = (_N - split) // _NW
    nvr = rpw // _L

    def _sc_body(x_hbm, t_hbm, q_hbm, out_hbm,
                 q_v, idx_v, qs_v, xb0, tb0, xb1, tb1, xb2, tb2, accb,
                 sx0, st0, sx1, st1, sx2, st2):
        wid = lax.axis_index("s") * _NC + lax.axis_index("c")
        base = split + wid * rpw
        pltpu.sync_copy(q_hbm.at[pl.ds(base, rpw)], q_v)

        def _pad(j, _):
            sl = pl.ds(j * _L, _L)
            idx_v[sl] = jnp.full((_L,), base, jnp.int32)
            qs_v[sl] = jnp.zeros((_L,), jnp.int32)
            return 0

        lax.fori_loop(0, nvr + 1, _pad, 0)

        def _compact(j, cnt):
            qv = q_v[pl.ds(j * _L, _L)]
            m = qv != 0
            mi = jnp.where(m, 1, 0)
            pos = jnp.maximum(cnt + plsc.cumsum(mi) - 1, 0)
            rows = lax.iota(jnp.int32, _L) + (base + j * _L)
            plsc.store_scatter(idx_v, [pos], rows, mask=m)
            plsc.store_scatter(qs_v, [pos], qv, mask=m)
            return cnt + jnp.sum(mi)

        cnt = lax.fori_loop(0, nvr, _compact, jnp.int32(0))

        nch = (cnt + _K - 1) // _K
        ngrp = (nch + _NB - 1) // _NB

        bufs = ((xb0, tb0, sx0, st0), (xb1, tb1, sx1, st1), (xb2, tb2, sx2, st2))

        def issue(c, b):
            off = pl.multiple_of(c * _K, _K)
            idxs = idx_v.at[pl.ds(off, _K)]
            xb, tb, sx, st = bufs[b]
            pltpu.async_copy(x_hbm.at[idxs], xb, sx)
            pltpu.async_copy(t_hbm.at[idxs], tb, st)

        def wait(b):
            xb, tb, sx, st = bufs[b]
            dummy = idx_v.at[pl.ds(0, _K)]
            pltpu.make_async_copy(x_hbm.at[dummy], xb, sx).wait()
            pltpu.make_async_copy(t_hbm.at[dummy], tb, st).wait()

        def compute(c, b, acc):
            off = pl.multiple_of(c * _K, _K)
            xb, tb, _, _ = bufs[b]

            def row(k, a0):
                qb = plsc.load_gather(qs_v, [jnp.full((_L,), off + k, jnp.int32)])
                m1 = qb == 1
                g2 = jnp.where(qb == 2, 1.0, 0.0)

                def inner(v, a):
                    for u in range(_UNROLL):
                        sl = pl.ds((v * _UNROLL + u) * _L, _L)
                        x = xb[k, sl]
                        t = tb[k, sl]
                        d = x - t
                        w = jnp.where(m1, t, g2)
                        g = w * d
                        a = a + g * g
                    return a

                return lax.fori_loop(0, _VPR // _UNROLL, inner, a0)

            return lax.fori_loop(0, _K, row, acc)

        @pl.when(nch < _NB)
        def _zero_bufs():
            zk = jnp.zeros((_L,), jnp.float32)
            for xb, tb, _, _ in bufs:
                def _z(w, _, _xb=xb, _tb=tb):
                    k = w // _VPR
                    sl = pl.ds((w % _VPR) * _L, _L)
                    _xb[k, sl] = zk
                    _tb[k, sl] = zk
                    return 0

                lax.fori_loop(0, _K * _VPR, _z, 0)

        for b in range(_NB):
            @pl.when(b < nch)
            def _prime(b=b):
                issue(b, b)

        def group(i, acc):
            c0 = i * _NB
            for b in range(_NB):
                c = c0 + b

                @pl.when(c < nch)
                def _w(b=b):
                    wait(b)

                acc = compute(c, b, acc)

                @pl.when(c + _NB < nch)
                def _i(c=c, b=b):
                    issue(c + _NB, b)
            return acc

        acc = lax.fori_loop(0, ngrp, group, jnp.zeros((_L,), jnp.float32))

        accb[...] = acc
        pltpu.sync_copy(accb, out_hbm.at[wid])

    return _sc_body, rpw


def _sc_partials(input, target, q, split):
    body, rpw = _make_sc_body(split)
    ipad = rpw + _L
    mesh = plsc.VectorSubcoreMesh(core_axis_name="c", subcore_axis_name="s")
    f = functools.partial(
        pl.kernel,
        mesh=mesh,
        compiler_params=pltpu.CompilerParams(needs_layout_passes=False),
        out_type=jax.ShapeDtypeStruct((_NW, _L), jnp.float32),
        scratch_types=[
            pltpu.VMEM((rpw,), jnp.int32),
            pltpu.VMEM((ipad,), jnp.int32),
            pltpu.VMEM((ipad,), jnp.int32),
            pltpu.VMEM((_K, _D), jnp.float32),
            pltpu.VMEM((_K, _D), jnp.float32),
            pltpu.VMEM((_K, _D), jnp.float32),
            pltpu.VMEM((_K, _D), jnp.float32),
            pltpu.VMEM((_K, _D), jnp.float32),
            pltpu.VMEM((_K, _D), jnp.float32),
            pltpu.VMEM((_L,), jnp.float32),
            pltpu.SemaphoreType.DMA,
            pltpu.SemaphoreType.DMA,
            pltpu.SemaphoreType.DMA,
            pltpu.SemaphoreType.DMA,
            pltpu.SemaphoreType.DMA,
            pltpu.SemaphoreType.DMA,
        ],
    )(body)
    return f(input, target, q)


def _tc_body(q_ref, x_ref, t_ref, out_ref):
    i = pl.program_id(0)

    @pl.when(i == 0)
    def _init():
        out_ref[...] = jnp.zeros_like(out_ref)

    x = x_ref[...]
    t = t_ref[...]
    qb = q_ref[...]
    d = x - t
    w = jnp.where(qb == 1, t, jnp.where(qb == 2, 1.0, 0.0).astype(jnp.float32))
    g = w * d
    out_ref[...] += jnp.sum(g * g).reshape(1, 1)


def _tc_sum(input, target, q, split):
    q2 = q[:, None]
    return pl.pallas_call(
        _tc_body,
        grid=(split // _TCBLK,),
        in_specs=[
            pl.BlockSpec((_TCBLK, 1), lambda i: (i, 0)),
            pl.BlockSpec((_TCBLK, _D), lambda i: (i, 0)),
            pl.BlockSpec((_TCBLK, _D), lambda i: (i, 0)),
        ],
        out_specs=pl.BlockSpec((1, 1), lambda i: (0, 0)),
        out_shape=jax.ShapeDtypeStruct((1, 1), jnp.float32),
    )(q2, input, target)


def kernel(input, target, q):
    sc = _sc_partials(input, target, q, _S)
    tc = _tc_sum(input, target, q, _S)
    return (tc[0, 0] + jnp.sum(sc)) / (_N * _D)


# speedup vs baseline: 1.0953x; 1.0895x over previous
import functools

import jax
import jax.numpy as jnp
from jax import lax
from jax.experimental import pallas as pl
from jax.experimental.pallas import tpu as pltpu
from jax.experimental.pallas import tpu_sc as plsc

_N, _D = 8192, 2048
_NC, _NS, _L = 2, 16, 16
_NW = _NC * _NS
_K = 8
_NB = 3
_UNROLL = 8
_VPR = _D // _L

_S = 4096
_TCBLK = 512


def _make_sc_body(split):
    rpw = (_N - split) // _NW
    nvr = rpw // _L

    def _sc_body(x_hbm, t_hbm, q_hbm, out_hbm,
                 q_v, idx_v, qs_v, xb0, tb0, xb1, tb1, xb2, tb2, accb,
                 sx0, st0, sx1, st1, sx2, st2):
        wid = lax.axis_index("s") * _NC + lax.axis_index("c")
        base = split + wid * rpw
        pltpu.sync_copy(q_hbm.at[pl.ds(base, rpw)], q_v)

        def _pad(j, _):
            sl = pl.ds(j * _L, _L)
            idx_v[sl] = jnp.full((_L,), base, jnp.int32)
            qs_v[sl] = jnp.zeros((_L,), jnp.int32)
            return 0

        lax.fori_loop(0, nvr + 1, _pad, 0)

        def _compact(j, cnt):
            qv = q_v[pl.ds(j * _L, _L)]
            m = qv != 0
            mi = jnp.where(m, 1, 0)
            pos = jnp.maximum(cnt + plsc.cumsum(mi) - 1, 0)
            rows = lax.iota(jnp.int32, _L) + (base + j * _L)
            plsc.store_scatter(idx_v, [pos], rows, mask=m)
            plsc.store_scatter(qs_v, [pos], qv, mask=m)
            return cnt + jnp.sum(mi)

        cnt = lax.fori_loop(0, nvr, _compact, jnp.int32(0))

        nch = (cnt + _K - 1) // _K
        ngrp = (nch + _NB - 1) // _NB

        bufs = ((xb0, tb0, sx0, st0), (xb1, tb1, sx1, st1), (xb2, tb2, sx2, st2))

        def issue(c, b):
            off = pl.multiple_of(c * _K, _K)
            idxs = idx_v.at[pl.ds(off, _K)]
            xb, tb, sx, st = bufs[b]
            pltpu.async_copy(x_hbm.at[idxs], xb, sx)
            pltpu.async_copy(t_hbm.at[idxs], tb, st)

        def wait(b):
            xb, tb, sx, st = bufs[b]
            dummy = idx_v.at[pl.ds(0, _K)]
            pltpu.make_async_copy(x_hbm.at[dummy], xb, sx).wait()
            pltpu.make_async_copy(t_hbm.at[dummy], tb, st).wait()

        def compute(c, b, acc):
            off = pl.multiple_of(c * _K, _K)
            xb, tb, _, _ = bufs[b]

            def row(k, a0):
                qb = plsc.load_gather(qs_v, [jnp.full((_L,), off + k, jnp.int32)])
                m1 = qb == 1
                g2 = jnp.where(qb == 2, 1.0, 0.0)

                def inner(v, a):
                    for u in range(_UNROLL):
                        sl = pl.ds((v * _UNROLL + u) * _L, _L)
                        x = xb[k, sl]
                        t = tb[k, sl]
                        d = x - t
                        w = jnp.where(m1, t, g2)
                        g = w * d
                        a = a + g * g
                    return a

                return lax.fori_loop(0, _VPR // _UNROLL, inner, a0)

            return lax.fori_loop(0, _K, row, acc)

        @pl.when(nch < _NB)
        def _zero_bufs():
            zk = jnp.zeros((_L,), jnp.float32)
            for xb, tb, _, _ in bufs:
                def _z(w, _, _xb=xb, _tb=tb):
                    k = w // _VPR
                    sl = pl.ds((w % _VPR) * _L, _L)
                    _xb[k, sl] = zk
                    _tb[k, sl] = zk
                    return 0

                lax.fori_loop(0, _K * _VPR, _z, 0)

        for b in range(_NB):
            @pl.when(b < nch)
            def _prime(b=b):
                issue(b, b)

        def group(i, acc):
            c0 = i * _NB
            for b in range(_NB):
                c = c0 + b

                @pl.when(c < nch)
                def _w(b=b):
                    wait(b)

                acc = compute(c, b, acc)

                @pl.when(c + _NB < nch)
                def _i(c=c, b=b):
                    issue(c + _NB, b)
            return acc

        acc = lax.fori_loop(0, ngrp, group, jnp.zeros((_L,), jnp.float32))

        accb[...] = acc
        pltpu.sync_copy(accb, out_hbm.at[wid])

    return _sc_body, rpw


def _sc_partials(input, target, q, split):
    body, rpw = _make_sc_body(split)
    ipad = rpw + _L
    mesh = plsc.VectorSubcoreMesh(core_axis_name="c", subcore_axis_name="s")
    f = functools.partial(
        pl.kernel,
        mesh=mesh,
        compiler_params=pltpu.CompilerParams(needs_layout_passes=False),
        out_type=jax.ShapeDtypeStruct((_NW, _L), jnp.float32),
        scratch_types=[
            pltpu.VMEM((rpw,), jnp.int32),
            pltpu.VMEM((ipad,), jnp.int32),
            pltpu.VMEM((ipad,), jnp.int32),
            pltpu.VMEM((_K, _D), jnp.float32),
            pltpu.VMEM((_K, _D), jnp.float32),
            pltpu.VMEM((_K, _D), jnp.float32),
            pltpu.VMEM((_K, _D), jnp.float32),
            pltpu.VMEM((_K, _D), jnp.float32),
            pltpu.VMEM((_K, _D), jnp.float32),
            pltpu.VMEM((_L,), jnp.float32),
            pltpu.SemaphoreType.DMA,
            pltpu.SemaphoreType.DMA,
            pltpu.SemaphoreType.DMA,
            pltpu.SemaphoreType.DMA,
            pltpu.SemaphoreType.DMA,
            pltpu.SemaphoreType.DMA,
        ],
    )(body)
    return f(input, target, q)


def _tc_body(q_ref, x_ref, t_ref, out_ref):
    i = pl.program_id(0)

    @pl.when(i == 0)
    def _init():
        out_ref[...] = jnp.zeros_like(out_ref)

    x = x_ref[...]
    t = t_ref[...]
    qb = q_ref[...]
    d = x - t
    w = jnp.where(qb == 1, t, jnp.where(qb == 2, 1.0, 0.0).astype(jnp.float32))
    g = w * d
    out_ref[...] += jnp.sum(g * g).reshape(1, 1)


def _tc_sum(input, target, q, split):
    q2 = q[:, None]
    return pl.pallas_call(
        _tc_body,
        grid=(split // _TCBLK,),
        in_specs=[
            pl.BlockSpec((_TCBLK, 1), lambda i: (i, 0)),
            pl.BlockSpec((_TCBLK, _D), lambda i: (i, 0)),
            pl.BlockSpec((_TCBLK, _D), lambda i: (i, 0)),
        ],
        out_specs=pl.BlockSpec((1, 1), lambda i: (0, 0)),
        out_shape=jax.ShapeDtypeStruct((1, 1), jnp.float32),
    )(q2, input, target)


def kernel(input, target, q):
    sc = _sc_partials(input, target, q, _S)
    tc = _tc_sum(input, target, q, _S)
    return (tc[0, 0] + jnp.sum(sc)) / (_N * _D)
